# Initial kernel scaffold; baseline (speedup 1.0000x reference)
import jax
import jax.numpy as jnp
from jax.experimental import pallas as pl


def kernel(x, edge_index, edge_weights, batch, W1, b1, gamma, beta, W2, b2, Wl, bl):
    raise NotImplementedError("write your pallas kernel here")


# speedup vs baseline: 10.0870x; 10.0870x over previous
import functools

import jax
import jax.numpy as jnp
from jax import lax
from jax.experimental import pallas as pl
from jax.experimental.pallas import tpu as pltpu
from jax.experimental.pallas import tpu_sc as plsc

N = 10000
E = 320000
D = 128
H = 128
O = 64
G = 8

NP = 10240
RP = NP // 16
NW = 32
EW = E // NW
C = 80
NCH = EW // C

_f32 = jnp.float32
_mesh = plsc.VectorSubcoreMesh(core_axis_name="c", subcore_axis_name="s")


def _zero16():
    return jnp.zeros((16,), _f32)


@functools.partial(
    pl.kernel,
    out_type=jax.ShapeDtypeStruct((2, NP), _f32),
    mesh=_mesh,
    scratch_types=[
        pltpu.VMEM((C,), jnp.int32),
        pltpu.VMEM((C,), _f32),
        pltpu.VMEM((RP,), _f32),
        pltpu.VMEM_SHARED((NP,), _f32),
    ],
)
def _deg_kernel(col_hbm, w_hbm, out_hbm, cibuf, wbuf, dbounce, dacc):
    cid = lax.axis_index("c")
    sid = lax.axis_index("s")
    gwid = sid * 2 + cid

    for i in range(C // 16):
        wbuf[pl.ds(i * 16, 16)] = _zero16()
    for k in range(RP // C):
        pltpu.sync_copy(wbuf, dacc.at[pl.ds(sid * RP + k * C, C)])
    plsc.subcore_barrier()

    def chunk(g, carry):
        base = gwid * EW + g * C
        pltpu.sync_copy(col_hbm.at[pl.ds(base, C)], cibuf)
        pltpu.sync_copy(w_hbm.at[pl.ds(base, C)], wbuf)
        pltpu.sync_copy(wbuf, dacc.at[cibuf], add=True)
        return carry

    lax.fori_loop(0, NCH, chunk, 0)
    plsc.subcore_barrier()

    pltpu.sync_copy(dacc.at[pl.ds(sid * RP, RP)], dbounce)
    pltpu.sync_copy(dbounce, out_hbm.at[cid, pl.ds(sid * RP, RP)])


@functools.partial(
    pl.kernel,
    out_type=jax.ShapeDtypeStruct((2, NP, D), _f32),
    mesh=_mesh,
    scratch_types=[
        pltpu.VMEM((C,), jnp.int32),
        pltpu.VMEM((C,), jnp.int32),
        pltpu.VMEM((C,), _f32),
        pltpu.VMEM((C, D), _f32),
        pltpu.VMEM_SHARED((NP, D), _f32),
        pltpu.SemaphoreType.DMA,
    ],
)
def _msg_kernel(xs_hbm, row_hbm, col_hbm, w_hbm, out_hbm,
                ribuf, cibuf, wbuf, gbuf, acc, sem):
    cid = lax.axis_index("c")
    sid = lax.axis_index("s")
    gwid = sid * 2 + cid

    def zrow(r, carry):
        for k in range(D // 16):
            gbuf[r, pl.ds(k * 16, 16)] = _zero16()
        return carry

    lax.fori_loop(0, C, zrow, 0)
    for k in range(RP // C):
        pltpu.sync_copy(gbuf, acc.at[pl.ds(sid * RP + k * C, C)])
    plsc.subcore_barrier()

    def chunk(g, carry):
        base = gwid * EW + g * C
        pltpu.sync_copy(row_hbm.at[pl.ds(base, C)], ribuf)
        pltpu.sync_copy(col_hbm.at[pl.ds(base, C)], cibuf)
        pltpu.sync_copy(w_hbm.at[pl.ds(base, C)], wbuf)
        pltpu.async_copy(xs_hbm.at[ribuf], gbuf, sem).wait()

        def scale(grp, c2):
            wv = wbuf[pl.ds(grp * 16, 16)]
            for j in range(16):
                e = grp * 16 + j
                ws = wv[j]
                for k in range(D // 16):
                    gbuf[e, pl.ds(k * 16, 16)] = gbuf[e, pl.ds(k * 16, 16)] * ws
            return c2

        lax.fori_loop(0, C // 16, scale, 0)
        pltpu.sync_copy(gbuf, acc.at[cibuf], add=True)
        return carry

    lax.fori_loop(0, NCH, chunk, 0)
    plsc.subcore_barrier()

    for k in range(RP // C):
        off = sid * RP + k * C
        pltpu.sync_copy(acc.at[pl.ds(off, C)], gbuf)
        pltpu.sync_copy(gbuf, out_hbm.at[cid, pl.ds(off, C)])


def _t1_body(x_ref, w1_ref, d0_ref, d1_ref, xs_ref, dinv_ref):
    deg = d0_ref[...] + d1_ref[...] + 1.0
    dinv = lax.rsqrt(deg)
    xw = jnp.dot(x_ref[...], w1_ref[...], preferred_element_type=_f32,
                 precision=lax.Precision.HIGHEST)
    xs_ref[...] = xw * dinv
    dinv_ref[...] = dinv


def _t2_body(accp_ref, xs_ref, dinv_ref, b1_ref, gamma_ref, beta_ref, w2_ref,
             hs2_ref):
    acc = accp_ref[0, :N, :] + accp_ref[1, :N, :]
    dinv = dinv_ref[...]
    pre = (acc + xs_ref[...]) * dinv + b1_ref[...]
    mean = jnp.mean(pre, axis=0, keepdims=True)
    cen = pre - mean
    var = jnp.mean(cen * cen, axis=0, keepdims=True)
    h = cen * lax.rsqrt(var + 1e-5) * gamma_ref[...] + beta_ref[...]
    h = jnp.maximum(h, 0.0)
    hs2_ref[...] = jnp.dot(h, w2_ref[...], preferred_element_type=_f32,
                           precision=lax.Precision.HIGHEST) * dinv


def _t3_body(acc2_ref, hs2_ref, dinv_ref, b2_ref, batch_ref, wl_ref, bl_ref,
             out_ref):
    acc = acc2_ref[0, :N, :] + acc2_ref[1, :N, :]
    h2 = (acc + hs2_ref[...]) * dinv_ref[...] + b2_ref[...]
    onehot = (batch_ref[...] == lax.broadcasted_iota(jnp.int32, (N, G), 1))
    onehot = onehot.astype(_f32)
    pooled = lax.dot_general(onehot, h2, (((0,), (0,)), ((), ())),
                             preferred_element_type=_f32,
                             precision=lax.Precision.HIGHEST)
    out_ref[...] = jnp.dot(pooled, wl_ref[...], preferred_element_type=_f32,
                           precision=lax.Precision.HIGHEST) + bl_ref[...]


_t1 = pl.pallas_call(
    _t1_body,
    out_shape=[jax.ShapeDtypeStruct((N, D), _f32),
               jax.ShapeDtypeStruct((N, 1), _f32)],
)

_t2 = pl.pallas_call(
    _t2_body,
    out_shape=jax.ShapeDtypeStruct((N, H), _f32),
)

_t3 = pl.pallas_call(
    _t3_body,
    out_shape=jax.ShapeDtypeStruct((G, O), _f32),
)


def kernel(x, edge_index, edge_weights, batch, W1, b1, gamma, beta, W2, b2,
           Wl, bl):
    row = edge_index[0]
    col = edge_index[1]

    degp = _deg_kernel(col, edge_weights)
    d0 = degp[0, :N].reshape(N, 1)
    d1 = degp[1, :N].reshape(N, 1)

    xs, dinv = _t1(x, W1, d0, d1)
    accp = _msg_kernel(xs, row, col, edge_weights)
    hs2 = _t2(accp, xs, dinv, b1.reshape(1, H), gamma.reshape(1, H),
              beta.reshape(1, H), W2)
    acc2p = _msg_kernel(hs2, row, col, edge_weights)
    out = _t3(acc2p, hs2, dinv, b2.reshape(1, H), batch.reshape(N, 1), Wl,
              bl.reshape(1, O))
    return out


# speedup vs baseline: 25.7582x; 2.5536x over previous
import functools

import jax
import jax.numpy as jnp
from jax import lax
from jax.experimental import pallas as pl
from jax.experimental.pallas import tpu as pltpu
from jax.experimental.pallas import tpu_sc as plsc

N = 10000
E = 320000
D = 128
H = 128
O = 64
G = 8

NP = 10240
RP = NP // 16
NW = 32
EW = E // NW
C = 80
NCH = EW // C

_f32 = jnp.float32
_mesh = plsc.VectorSubcoreMesh(core_axis_name="c", subcore_axis_name="s")


def _zero16():
    return jnp.zeros((16,), _f32)


@functools.partial(
    pl.kernel,
    out_type=jax.ShapeDtypeStruct((2, NP), _f32),
    mesh=_mesh,
    scratch_types=[
        pltpu.VMEM((NCH, C), jnp.int32),
        pltpu.VMEM((NCH, C), _f32),
        pltpu.VMEM((RP,), _f32),
        pltpu.VMEM_SHARED((NP,), _f32),
    ],
)
def _deg_kernel(col_hbm, w_hbm, out_hbm, cib, wb, dbounce, dacc):
    cid = lax.axis_index("c")
    sid = lax.axis_index("s")
    gwid = sid * 2 + cid

    for i in range(RP // 16):
        dbounce[pl.ds(i * 16, 16)] = _zero16()
    pltpu.sync_copy(dbounce, dacc.at[pl.ds(sid * RP, RP)])
    pltpu.sync_copy(col_hbm.at[gwid], cib)
    pltpu.sync_copy(w_hbm.at[gwid], wb)
    plsc.subcore_barrier()

    def chunk(g, carry):
        pltpu.sync_copy(wb.at[g], dacc.at[cib.at[g]], add=True)
        return carry

    lax.fori_loop(0, NCH, chunk, 0)
    plsc.subcore_barrier()

    pltpu.sync_copy(dacc.at[pl.ds(sid * RP, RP)], dbounce)
    pltpu.sync_copy(dbounce, out_hbm.at[cid, pl.ds(sid * RP, RP)])


@functools.partial(
    pl.kernel,
    out_type=jax.ShapeDtypeStruct((2, NP, D), _f32),
    mesh=_mesh,
    scratch_types=[
        pltpu.VMEM((3, C), jnp.int32),
        pltpu.VMEM((3, C), jnp.int32),
        pltpu.VMEM((3, C), jnp.int32),
        pltpu.VMEM((3, C), jnp.int32),
        pltpu.VMEM((C, D), _f32),
        pltpu.VMEM((C, D), _f32),
        pltpu.VMEM_SHARED((NP, D), _f32),
        pltpu.SemaphoreType.DMA,
        pltpu.SemaphoreType.DMA,
        pltpu.SemaphoreType.DMA,
        pltpu.SemaphoreType.DMA,
        pltpu.SemaphoreType.DMA,
        pltpu.SemaphoreType.DMA,
    ],
)
def _msg_kernel(xs_hbm, pk_hbm, out_hbm,
                pk0, pk1, pk2, pk3, gb0, gb1, acc,
                ps0, ps1, ps2, ps3, gs0, gs1):
    cid = lax.axis_index("c")
    sid = lax.axis_index("s")
    gwid = sid * 2 + cid
    pkb = (pk0, pk1, pk2, pk3)
    pks = (ps0, ps1, ps2, ps3)
    gbufs = (gb0, gb1)
    gsems = (gs0, gs1)

    def zrow(r, carry):
        for k in range(D // 16):
            gb0[r, pl.ds(k * 16, 16)] = _zero16()
        return carry

    lax.fori_loop(0, C, zrow, 0)
    for k in range(RP // C):
        pltpu.sync_copy(gb0, acc.at[pl.ds(sid * RP + k * C, C)])
    plsc.subcore_barrier()

    def start_pk(q, r):
        pltpu.async_copy(pk_hbm.at[gwid, q], pkb[r], pks[r])

    def wait_pk(q, r):
        pltpu.make_async_copy(pk_hbm.at[gwid, q], pkb[r], pks[r]).wait()

    def start_ga(q, r, b):
        pltpu.async_copy(xs_hbm.at[pkb[r].at[0]], gbufs[b], gsems[b])

    def wait_ga(q, r, b):
        pltpu.make_async_copy(xs_hbm.at[pkb[r].at[0]], gbufs[b],
                              gsems[b]).wait()

    def process(q, r, b):
        gb = gbufs[b]
        pk = pkb[r]

        def scale(grp, c2):
            wv = lax.bitcast_convert_type(pk[2, pl.ds(grp * 16, 16)], _f32)
            for jj in range(16):
                e = grp * 16 + jj
                ws = wv[jj]
                for k in range(D // 16):
                    gb[e, pl.ds(k * 16, 16)] = gb[e, pl.ds(k * 16, 16)] * ws
            return c2

        lax.fori_loop(0, C // 16, scale, 0)
        pltpu.sync_copy(gb, acc.at[pk.at[1]], add=True)

    start_pk(0, 0)
    start_pk(1, 1)
    start_pk(2, 2)
    wait_pk(0, 0)
    start_ga(0, 0, 0)
    wait_pk(1, 1)
    start_ga(1, 1, 1)

    def quad(go, carry):
        g = go * 4
        for j in range(4):
            q = g + j
            b = j % 2
            wait_ga(q, j, b)
            process(q, j, b)

            @pl.when(q + 3 < NCH)
            def _():
                start_pk(q + 3, (j + 3) % 4)

            @pl.when(q + 2 < NCH)
            def _():
                wait_pk(q + 2, (j + 2) % 4)
                start_ga(q + 2, (j + 2) % 4, b)

        return carry

    lax.fori_loop(0, NCH // 4, quad, 0)
    wait_ga(NCH - 1, 0, 0)
    process(NCH - 1, 0, 0)
    plsc.subcore_barrier()

    for k in range(RP // C):
        off = sid * RP + k * C
        pltpu.sync_copy(acc.at[pl.ds(off, C)], gb0)
        pltpu.sync_copy(gb0, out_hbm.at[cid, pl.ds(off, C)])


def _t1_body(x_ref, w1_ref, d0_ref, d1_ref, xs_ref, dinv_ref):
    deg = d0_ref[...] + d1_ref[...] + 1.0
    dinv = lax.rsqrt(deg)
    xw = jnp.dot(x_ref[...], w1_ref[...], preferred_element_type=_f32,
                 precision=lax.Precision.HIGHEST)
    xs_ref[...] = xw * dinv
    dinv_ref[...] = dinv


def _t2_body(accp_ref, xs_ref, dinv_ref, b1_ref, gamma_ref, beta_ref, w2_ref,
             hs2_ref):
    acc = accp_ref[0, :N, :] + accp_ref[1, :N, :]
    dinv = dinv_ref[...]
    pre = (acc + xs_ref[...]) * dinv + b1_ref[...]
    mean = jnp.mean(pre, axis=0, keepdims=True)
    cen = pre - mean
    var = jnp.mean(cen * cen, axis=0, keepdims=True)
    h = cen * lax.rsqrt(var + 1e-5) * gamma_ref[...] + beta_ref[...]
    h = jnp.maximum(h, 0.0)
    hs2_ref[...] = jnp.dot(h, w2_ref[...], preferred_element_type=_f32,
                           precision=lax.Precision.HIGHEST) * dinv


def _t3_body(acc2_ref, hs2_ref, dinv_ref, b2_ref, batch_ref, wl_ref, bl_ref,
             out_ref):
    acc = acc2_ref[0, :N, :] + acc2_ref[1, :N, :]
    h2 = (acc + hs2_ref[...]) * dinv_ref[...] + b2_ref[...]
    onehot = (batch_ref[...] == lax.broadcasted_iota(jnp.int32, (N, G), 1))
    onehot = onehot.astype(_f32)
    pooled = lax.dot_general(onehot, h2, (((0,), (0,)), ((), ())),
                             preferred_element_type=_f32,
                             precision=lax.Precision.HIGHEST)
    out_ref[...] = jnp.dot(pooled, wl_ref[...], preferred_element_type=_f32,
                           precision=lax.Precision.HIGHEST) + bl_ref[...]


_t1 = pl.pallas_call(
    _t1_body,
    out_shape=[jax.ShapeDtypeStruct((N, D), _f32),
               jax.ShapeDtypeStruct((N, 1), _f32)],
)

_t2 = pl.pallas_call(
    _t2_body,
    out_shape=jax.ShapeDtypeStruct((N, H), _f32),
)

_t3 = pl.pallas_call(
    _t3_body,
    out_shape=jax.ShapeDtypeStruct((G, O), _f32),
)


def kernel(x, edge_index, edge_weights, batch, W1, b1, gamma, beta, W2, b2,
           Wl, bl):
    row = edge_index[0].reshape(NW, NCH, C)
    col = edge_index[1].reshape(NW, NCH, C)
    ew = edge_weights.reshape(NW, NCH, C)
    wbits = lax.bitcast_convert_type(ew, jnp.int32)
    pk = jnp.stack([row, col, wbits], axis=2)

    degp = _deg_kernel(col, ew)
    d0 = degp[0, :N].reshape(N, 1)
    d1 = degp[1, :N].reshape(N, 1)

    xs, dinv = _t1(x, W1, d0, d1)
    accp = _msg_kernel(xs, pk)
    hs2 = _t2(accp, xs, dinv, b1.reshape(1, H), gamma.reshape(1, H),
              beta.reshape(1, H), W2)
    acc2p = _msg_kernel(hs2, pk)
    out = _t3(acc2p, hs2, dinv, b2.reshape(1, H), batch.reshape(N, 1), Wl,
              bl.reshape(1, O))
    return out


# speedup vs baseline: 27.0178x; 1.0489x over previous
import functools

import jax
import jax.numpy as jnp
from jax import lax
from jax.experimental import pallas as pl
from jax.experimental.pallas import tpu as pltpu
from jax.experimental.pallas import tpu_sc as plsc

N = 10000
E = 320000
D = 128
H = 128
O = 64
G = 8

NP = 10240
RP = NP // 16
NW = 32
EW = E // NW
C = 80
NCH = EW // C

_f32 = jnp.float32
_mesh = plsc.VectorSubcoreMesh(core_axis_name="c", subcore_axis_name="s")


def _zero16():
    return jnp.zeros((16,), _f32)


@functools.partial(
    pl.kernel,
    out_type=jax.ShapeDtypeStruct((2, NP), _f32),
    mesh=_mesh,
    scratch_types=[
        pltpu.VMEM((NCH, C), jnp.int32),
        pltpu.VMEM((NCH, C), _f32),
        pltpu.VMEM((RP,), _f32),
        pltpu.VMEM_SHARED((NP,), _f32),
    ],
)
def _deg_kernel(col_hbm, w_hbm, out_hbm, cib, wb, dbounce, dacc):
    cid = lax.axis_index("c")
    sid = lax.axis_index("s")
    gwid = sid * 2 + cid

    for i in range(RP // 16):
        dbounce[pl.ds(i * 16, 16)] = _zero16()
    pltpu.sync_copy(dbounce, dacc.at[pl.ds(sid * RP, RP)])
    pltpu.sync_copy(col_hbm.at[gwid], cib)
    pltpu.sync_copy(w_hbm.at[gwid], wb)
    plsc.subcore_barrier()

    def chunk(g, carry):
        pltpu.sync_copy(wb.at[g], dacc.at[cib.at[g]], add=True)
        return carry

    lax.fori_loop(0, NCH, chunk, 0)
    plsc.subcore_barrier()

    pltpu.sync_copy(dacc.at[pl.ds(sid * RP, RP)], dbounce)
    pltpu.sync_copy(dbounce, out_hbm.at[cid, pl.ds(sid * RP, RP)])


_NGB = 3
_NPK = 6
_UNR = 6
_NTL = NCH - (NCH // _UNR) * _UNR


@functools.partial(
    pl.kernel,
    out_type=jax.ShapeDtypeStruct((2, NP, D), _f32),
    mesh=_mesh,
    scratch_types=(
        [pltpu.VMEM((3, C), jnp.int32)] * _NPK
        + [pltpu.VMEM((C, D), _f32)] * _NGB
        + [pltpu.VMEM_SHARED((NP, D), _f32)]
        + [pltpu.SemaphoreType.DMA] * (_NPK + 2 * _NGB)
    ),
)
def _msg_kernel(xs_hbm, pk_hbm, out_hbm, *refs):
    pkb = refs[:_NPK]
    gbufs = refs[_NPK:_NPK + _NGB]
    acc = refs[_NPK + _NGB]
    sems = refs[_NPK + _NGB + 1:]
    pks = sems[:_NPK]
    gsems = sems[_NPK:_NPK + _NGB]
    ssems = sems[_NPK + _NGB:]

    cid = lax.axis_index("c")
    sid = lax.axis_index("s")
    gwid = sid * 2 + cid
    gb0 = gbufs[0]

    def zrow(r, carry):
        for k in range(D // 16):
            gb0[r, pl.ds(k * 16, 16)] = _zero16()
        return carry

    lax.fori_loop(0, C, zrow, 0)
    for k in range(RP // C):
        pltpu.sync_copy(gb0, acc.at[pl.ds(sid * RP + k * C, C)])
    plsc.subcore_barrier()

    def start_pk(q, r):
        pltpu.async_copy(pk_hbm.at[gwid, q], pkb[r], pks[r])

    def wait_pk(q, r):
        pltpu.make_async_copy(pk_hbm.at[gwid, q], pkb[r], pks[r]).wait()

    def start_ga(q, r, b):
        pltpu.async_copy(xs_hbm.at[pkb[r].at[0]], gbufs[b], gsems[b])

    def wait_ga(q, r, b):
        pltpu.make_async_copy(xs_hbm.at[pkb[r].at[0]], gbufs[b],
                              gsems[b]).wait()

    def start_sc(q, r, b):
        pltpu.async_copy(gbufs[b], acc.at[pkb[r].at[1]], ssems[b], add=True)

    def wait_sc(q, r, b):
        pltpu.make_async_copy(gbufs[b], acc.at[pkb[r].at[1]],
                              ssems[b]).wait()

    def scale(q, r, b):
        gb = gbufs[b]
        pk = pkb[r]

        def body(grp, c2):
            wv = lax.bitcast_convert_type(pk[2, pl.ds(grp * 16, 16)], _f32)
            for jj in range(16):
                e = grp * 16 + jj
                ws = wv[jj]
                for k in range(D // 16):
                    gb[e, pl.ds(k * 16, 16)] = gb[e, pl.ds(k * 16, 16)] * ws
            return c2

        lax.fori_loop(0, C // 16, body, 0)

    def step(q, j, cond):
        b = j % _NGB
        r = j % _NPK
        wait_ga(q, r, b)
        scale(q, r, b)
        start_sc(q, r, b)
        cond(q >= 1, lambda: wait_sc(q - 1, (j - 1) % _NPK, (j - 1) % _NGB))
        cond(q + 2 < NCH,
             lambda: (wait_pk(q + 2, (j + 2) % _NPK),
                      start_ga(q + 2, (j + 2) % _NPK, (j + 2) % _NGB)))
        cond(q + 3 < NCH, lambda: start_pk(q + 3, (j + 3) % _NPK))

    start_pk(0, 0)
    start_pk(1, 1)
    start_pk(2, 2)
    wait_pk(0, 0)
    start_ga(0, 0, 0)
    wait_pk(1, 1)
    start_ga(1, 1, 1)

    def traced_cond(pred, fn):
        pl.when(pred)(lambda: (fn(), None)[1])

    def six(go, carry):
        g = go * _UNR
        for j in range(_UNR):
            step(g + j, j, traced_cond)
        return carry

    nfull = NCH // _UNR
    lax.fori_loop(0, nfull, six, 0)

    def static_cond(pred, fn):
        if pred:
            fn()

    for j in range(_NTL):
        step(nfull * _UNR + j, j, static_cond)

    qlast = NCH - 1
    wait_sc(qlast, (qlast % _UNR) % _NPK, (qlast % _UNR) % _NGB)
    plsc.subcore_barrier()

    for k in range(RP // C):
        off = sid * RP + k * C
        pltpu.sync_copy(acc.at[pl.ds(off, C)], gb0)
        pltpu.sync_copy(gb0, out_hbm.at[cid, pl.ds(off, C)])


def _t1_body(x_ref, w1_ref, d0_ref, d1_ref, xs_ref, dinv_ref):
    deg = d0_ref[...] + d1_ref[...] + 1.0
    dinv = lax.rsqrt(deg)
    xw = jnp.dot(x_ref[...], w1_ref[...], preferred_element_type=_f32,
                 precision=lax.Precision.HIGHEST)
    xs_ref[...] = xw * dinv
    dinv_ref[...] = dinv


def _t2_body(accp_ref, xs_ref, dinv_ref, b1_ref, gamma_ref, beta_ref, w2_ref,
             hs2_ref):
    acc = accp_ref[0, :N, :] + accp_ref[1, :N, :]
    dinv = dinv_ref[...]
    pre = (acc + xs_ref[...]) * dinv + b1_ref[...]
    mean = jnp.mean(pre, axis=0, keepdims=True)
    cen = pre - mean
    var = jnp.mean(cen * cen, axis=0, keepdims=True)
    h = cen * lax.rsqrt(var + 1e-5) * gamma_ref[...] + beta_ref[...]
    h = jnp.maximum(h, 0.0)
    hs2_ref[...] = jnp.dot(h, w2_ref[...], preferred_element_type=_f32,
                           precision=lax.Precision.HIGHEST) * dinv


def _t3_body(acc2_ref, hs2_ref, dinv_ref, b2_ref, batch_ref, wl_ref, bl_ref,
             out_ref):
    acc = acc2_ref[0, :N, :] + acc2_ref[1, :N, :]
    h2 = (acc + hs2_ref[...]) * dinv_ref[...] + b2_ref[...]
    onehot = (batch_ref[...] == lax.broadcasted_iota(jnp.int32, (N, G), 1))
    onehot = onehot.astype(_f32)
    pooled = lax.dot_general(onehot, h2, (((0,), (0,)), ((), ())),
                             preferred_element_type=_f32,
                             precision=lax.Precision.HIGHEST)
    out_ref[...] = jnp.dot(pooled, wl_ref[...], preferred_element_type=_f32,
                           precision=lax.Precision.HIGHEST) + bl_ref[...]


_t1 = pl.pallas_call(
    _t1_body,
    out_shape=[jax.ShapeDtypeStruct((N, D), _f32),
               jax.ShapeDtypeStruct((N, 1), _f32)],
)

_t2 = pl.pallas_call(
    _t2_body,
    out_shape=jax.ShapeDtypeStruct((N, H), _f32),
)

_t3 = pl.pallas_call(
    _t3_body,
    out_shape=jax.ShapeDtypeStruct((G, O), _f32),
)


def kernel(x, edge_index, edge_weights, batch, W1, b1, gamma, beta, W2, b2,
           Wl, bl):
    row = edge_index[0].reshape(NW, NCH, C)
    col = edge_index[1].reshape(NW, NCH, C)
    ew = edge_weights.reshape(NW, NCH, C)
    wbits = lax.bitcast_convert_type(ew, jnp.int32)
    pk = jnp.stack([row, col, wbits], axis=2)

    degp = _deg_kernel(col, ew)
    d0 = degp[0, :N].reshape(N, 1)
    d1 = degp[1, :N].reshape(N, 1)

    xs, dinv = _t1(x, W1, d0, d1)
    accp = _msg_kernel(xs, pk)
    hs2 = _t2(accp, xs, dinv, b1.reshape(1, H), gamma.reshape(1, H),
              beta.reshape(1, H), W2)
    acc2p = _msg_kernel(hs2, pk)
    out = _t3(acc2p, hs2, dinv, b2.reshape(1, H), batch.reshape(N, 1), Wl,
              bl.reshape(1, O))
    return out
